# baseline (speedup 1.0000x reference)
import numpy as np
import jax
import jax.numpy as jnp
from jax import lax
from jax.experimental import pallas as pl
from jax.experimental.pallas import tpu as pltpu
from jax.experimental.pallas import tpu_sc as plsc

NC = 2
NS = 16
NWORK = NC * NS
WVE = 80
FH = 64

_SC_PARAMS = pltpu.CompilerParams(use_tc_tiling_on_sc=False)
_BN_K = float(1.0 / np.sqrt(1.0 + 1e-5))


def _segsum_sc(feat, src3, dst3, n_nodes, with_deg):
  dt = feat.dtype
  VL = 32 if dt == jnp.bfloat16 else 16
  F = feat.shape[1]
  NWIN = src3.shape[-2]
  ZCH = n_nodes // NS

  out_types = [jax.ShapeDtypeStruct((NC, n_nodes, F), dt)]
  if with_deg:
    out_types.append(jax.ShapeDtypeStruct((NC, n_nodes), jnp.float32))

  scratch = [
      pltpu.VMEM((NWIN, WVE), jnp.int32),
      pltpu.VMEM((NWIN, WVE), jnp.int32),
      pltpu.VMEM((WVE, F), dt),
      pltpu.VMEM((WVE, F), dt),
      pltpu.VMEM((WVE, F), dt),
      pltpu.VMEM((WVE, F), dt),
      pltpu.VMEM_SHARED((n_nodes, F), dt),
      pltpu.SemaphoreType.DMA,
      pltpu.SemaphoreType.DMA,
  ]
  DCH = 640
  if with_deg:
    scratch.append(pltpu.VMEM((DCH,), jnp.float32))
    scratch.append(pltpu.VMEM_SHARED((n_nodes,), jnp.float32))

  mesh = plsc.VectorSubcoreMesh(core_axis_name="c", subcore_axis_name="s")

  def body(feat_hbm, src_hbm, dst_hbm, *rest):
    if with_deg:
      (out_hbm, deg_hbm, src_v, dst_v, buf0, buf1, buf2, buf3, acc, gsem,
       ssem, ones_v, dacc) = rest
    else:
      out_hbm, src_v, dst_v, buf0, buf1, buf2, buf3, acc, gsem, ssem = rest
    bufs = (buf0, buf1, buf2, buf3)
    cid = lax.axis_index("c")
    tid = lax.axis_index("s")

    wid = tid * NC + cid
    pltpu.sync_copy(src_hbm.at[wid], src_v)
    pltpu.sync_copy(dst_hbm.at[wid], dst_v)

    @pl.loop(0, WVE)
    def _(r):
      for cc in range(F // VL):
        buf0[r, pl.ds(cc * VL, VL)] = jnp.zeros((VL,), dt)

    base = tid * ZCH
    nfull, rem = ZCH // WVE, ZCH % WVE
    for k in range(nfull):
      pltpu.sync_copy(buf0, acc.at[pl.ds(base + k * WVE, WVE)])
    if rem:
      pltpu.sync_copy(buf0.at[pl.ds(0, rem)],
                      acc.at[pl.ds(base + nfull * WVE, rem)])

    if with_deg:
      dlast = n_nodes - (NS - 1) * DCH

      @pl.loop(0, DCH // 16)
      def _(r):
        ones_v[pl.ds(r * 16, 16)] = jnp.zeros((16,), jnp.float32)

      @pl.when(tid < NS - 1)
      def _():
        pltpu.sync_copy(ones_v, dacc.at[pl.ds(tid * DCH, DCH)])

      @pl.when(tid == NS - 1)
      def _():
        pltpu.sync_copy(ones_v.at[pl.ds(0, dlast)],
                        dacc.at[pl.ds((NS - 1) * DCH, dlast)])

      @pl.loop(0, DCH // 16)
      def _(r):
        ones_v[pl.ds(r * 16, 16)] = jnp.ones((16,), jnp.float32)

    plsc.subcore_barrier()

    def gstart(w, b):
      pltpu.async_copy(feat_hbm.at[src_v.at[w]], bufs[b], gsem)

    def gwait(b):
      pltpu.make_async_copy(feat_hbm.at[src_v.at[0]], bufs[b], gsem).wait()

    def sstart(w, b):
      pltpu.async_copy(bufs[b], acc.at[dst_v.at[w]], ssem, add=True)
      if with_deg:
        pltpu.sync_copy(ones_v.at[pl.ds(0, WVE)], dacc.at[dst_v.at[w]],
                        add=True)

    def swait(b):
      pltpu.make_async_copy(bufs[b], acc.at[dst_v.at[0]], ssem).wait()

    assert NWIN >= 8
    K = (NWIN - 8) // 4
    post0 = 4 + 4 * K

    gstart(0, 0)
    gstart(1, 1)
    for w in range(4):
      b = w % 4
      gwait(b)
      if w >= 2:
        swait(w - 2)
      sstart(w, b)
      gstart(w + 2, (w + 2) % 4)

    @pl.loop(4, post0, step=4)
    def _(j):
      for u in range(4):
        gwait(u)
        swait((u + 2) % 4)
        sstart(j + u, u)
        gstart(j + u + 2, (u + 2) % 4)

    for w in range(post0, NWIN):
      b = w % 4
      gwait(b)
      swait((w - 2) % 4)
      sstart(w, b)
      if w + 2 < NWIN:
        gstart(w + 2, (w + 2) % 4)
    swait((NWIN - 2) % 4)
    swait((NWIN - 1) % 4)

    plsc.subcore_barrier()

    pltpu.sync_copy(acc.at[pl.ds(base, ZCH)],
                    out_hbm.at[cid, pl.ds(base, ZCH)])
    if with_deg:
      @pl.when(tid == 0)
      def _():
        pltpu.sync_copy(dacc, deg_hbm.at[cid])

  k = pl.kernel(body, out_type=tuple(out_types), mesh=mesh,
                scratch_types=scratch, compiler_params=_SC_PARAMS)
  res = k(feat, src3, dst3)
  return res if with_deg else (res[0],)


def _tc_l0_pre(x, plm, ws, b, g, bb, BR=1000):
  n, F = x.shape

  def body(xr, pr, wsr, br, gr, bbr, outr):
    f32 = jnp.float32
    t = (jnp.dot(xr[...].astype(f32), wsr[...], preferred_element_type=f32)
         + br[...])
    outr[...] = ((t + pr[...]) * gr[...] * _BN_K + 2.0 * bbr[...])

  return pl.pallas_call(
      body, grid=(n // BR,),
      in_specs=[
          pl.BlockSpec((BR, F), lambda i: (i, 0)),
          pl.BlockSpec((BR, F), lambda i: (i, 0)),
          pl.BlockSpec((F, F), lambda i: (0, 0)),
          pl.BlockSpec((1, F), lambda i: (0, 0)),
          pl.BlockSpec((1, F), lambda i: (0, 0)),
          pl.BlockSpec((1, F), lambda i: (0, 0)),
      ],
      out_specs=pl.BlockSpec((BR, F), lambda i: (i, 0)),
      out_shape=jax.ShapeDtypeStruct((n, F), jnp.float32),
  )(x, plm, ws, b, g, bb)


def _tc_l0_post(u, p, dg, wn, g, BR=1000):
  n, F = u.shape

  def body(ur, ppr, dgr, wnr, gr, outr):
    f32 = jnp.float32
    dm = jnp.maximum(dgr[0] + dgr[1], 1.0)
    pm = (ppr[0].astype(f32) + ppr[1].astype(f32)) / dm
    v = jnp.dot(pm, wnr[...], preferred_element_type=f32) * gr[...] * _BN_K
    outr[...] = jnp.maximum(ur[...] + v, 0.0).astype(outr.dtype)

  return pl.pallas_call(
      body, grid=(n // BR,),
      in_specs=[
          pl.BlockSpec((BR, F), lambda i: (i, 0)),
          pl.BlockSpec((NC, BR, F), lambda i: (0, i, 0)),
          pl.BlockSpec((NC, BR, 1), lambda i: (0, i, 0)),
          pl.BlockSpec((F, F), lambda i: (0, 0)),
          pl.BlockSpec((1, F), lambda i: (0, 0)),
      ],
      out_specs=pl.BlockSpec((BR, F), lambda i: (i, 0)),
      out_shape=jax.ShapeDtypeStruct((n, F), jnp.bfloat16),
  )(u, p, dg, wn, g)


def _tc_layer01(x, plm, p, dg, ws, wn, b, g, bb, BR=1000):
  n, F = x.shape
  Fo = ws.shape[1]
  have_plm = plm is not None

  def body(*refs):
    if have_plm:
      xr, pr, ppr, dgr, wsr, wnr, br, gr, bbr, outr = refs
    else:
      xr, ppr, dgr, wsr, wnr, br, gr, bbr, outr = refs
    f32 = jnp.float32
    dm = jnp.maximum(dgr[0] + dgr[1], 1.0)
    pm = (ppr[0].astype(f32) + ppr[1].astype(f32)) / dm
    t = (jnp.dot(xr[...].astype(f32), wsr[...], preferred_element_type=f32)
         + jnp.dot(pm, wnr[...], preferred_element_type=f32)
         + br[...])
    t = t * gr[...] * _BN_K + bbr[...]
    if have_plm:
      t = t + pr[...] * gr[...] * _BN_K + bbr[...]
    outr[...] = jnp.maximum(t, 0.0).astype(outr.dtype)

  in_specs = [pl.BlockSpec((BR, F), lambda i: (i, 0))]
  if have_plm:
    in_specs.append(pl.BlockSpec((BR, F), lambda i: (i, 0)))
  in_specs += [
      pl.BlockSpec((NC, BR, F), lambda i: (0, i, 0)),
      pl.BlockSpec((NC, BR, 1), lambda i: (0, i, 0)),
      pl.BlockSpec((F, Fo), lambda i: (0, 0)),
      pl.BlockSpec((F, Fo), lambda i: (0, 0)),
      pl.BlockSpec((1, Fo), lambda i: (0, 0)),
      pl.BlockSpec((1, Fo), lambda i: (0, 0)),
      pl.BlockSpec((1, Fo), lambda i: (0, 0)),
  ]
  args = [x] + ([plm] if have_plm else []) + [p, dg, ws, wn, b, g, bb]
  return pl.pallas_call(
      body, grid=(n // BR,), in_specs=in_specs,
      out_specs=pl.BlockSpec((BR, Fo), lambda i: (i, 0)),
      out_shape=jax.ShapeDtypeStruct((n, Fo), x.dtype),
  )(*args)


def _tc_layer12a(x, p, dg, ws, wn, b, g, bb, wsp, wnp, bp, BR=1000):
  n, F = x.shape
  Fo = wsp.shape[1]

  def body(xr, ppr, dgr, wsr, wnr, br, gr, bbr, wspr, wnpr, bpr, sr, yr):
    f32 = jnp.float32
    dm = jnp.maximum(dgr[0] + dgr[1], 1.0)
    pm = (ppr[0].astype(f32) + ppr[1].astype(f32)) / dm
    t = (jnp.dot(xr[...].astype(f32), wsr[...], preferred_element_type=f32)
         + jnp.dot(pm, wnr[...], preferred_element_type=f32)
         + br[...])
    h2 = jnp.maximum(t * gr[...] * _BN_K + bbr[...], 0.0)
    sr[...] = (jnp.dot(h2, wspr[...], preferred_element_type=f32) + bpr[...])
    yr[...] = jnp.dot(h2, wnpr[...],
                      preferred_element_type=f32).astype(yr.dtype)

  full = lambda shape: pl.BlockSpec(shape, lambda i: tuple(0 for _ in shape))
  return pl.pallas_call(
      body, grid=(n // BR,),
      in_specs=[
          pl.BlockSpec((BR, F), lambda i: (i, 0)),
          pl.BlockSpec((NC, BR, F), lambda i: (0, i, 0)),
          pl.BlockSpec((NC, BR, 1), lambda i: (0, i, 0)),
          full((F, F)), full((F, F)), full((1, F)), full((1, F)),
          full((1, F)), full((F, Fo)), full((F, Fo)), full((1, Fo)),
      ],
      out_specs=[pl.BlockSpec((BR, Fo), lambda i: (i, 0)),
                 pl.BlockSpec((BR, Fo), lambda i: (i, 0))],
      out_shape=[jax.ShapeDtypeStruct((n, Fo), jnp.float32),
                 jax.ShapeDtypeStruct((n, Fo), x.dtype)],
  )(x, p, dg, ws, wn, b, g, bb, wsp, wnp, bp)


def _tc_layer2b(s, p, dg, BR=1000):
  n, Fo = s.shape

  def body(sr, ppr, dgr, outr):
    dm = jnp.maximum(dgr[0] + dgr[1], 1.0)
    outr[...] = sr[...] + (ppr[0].astype(jnp.float32)
                           + ppr[1].astype(jnp.float32)) / dm

  return pl.pallas_call(
      body, grid=(n // BR,),
      in_specs=[
          pl.BlockSpec((BR, Fo), lambda i: (i, 0)),
          pl.BlockSpec((NC, BR, Fo), lambda i: (0, i, 0)),
          pl.BlockSpec((NC, BR, 1), lambda i: (0, i, 0)),
      ],
      out_specs=pl.BlockSpec((BR, Fo), lambda i: (i, 0)),
      out_shape=jax.ShapeDtypeStruct((n, Fo), jnp.float32),
  )(s, p, dg)


def kernel(edge_index, LLM_feat, PLM_feat, W_self0, W_neigh0, b0,
           W_self1, W_neigh1, b1, W_self2, W_neigh2, b2,
           bn0_g, bn0_b, bn1_g, bn1_b):
  n, D = LLM_feat.shape
  E = edge_index.shape[1]
  C = W_self2.shape[1]
  EPW = E // NWORK
  NWIN = EPW // WVE
  assert E == NWORK * NWIN * WVE and n % NS == 0

  src1 = edge_index[0].astype(jnp.int32).reshape(NWORK, NWIN, WVE)
  dst1 = edge_index[1].astype(jnp.int32).reshape(NWORK, NWIN, WVE)

  row = lambda v: v.reshape(1, -1)

  llm_b = LLM_feat.astype(jnp.bfloat16)
  P0, Dg = _segsum_sc(llm_b, src1, dst1, n, True)
  dgc = Dg.reshape(NC, n, 1)
  U = _tc_l0_pre(llm_b, PLM_feat, W_self0, row(b0), row(bn0_g), row(bn0_b))
  h1 = _tc_l0_post(U, P0, dgc, W_neigh0, row(bn0_g))
  (P1,) = _segsum_sc(h1, src1, dst1, n, False)
  S, Y = _tc_layer12a(h1, P1, dgc, W_self1, W_neigh1,
                      row(b1), row(bn1_g), row(bn1_b),
                      jnp.pad(W_self2, ((0, 0), (0, FH - C))),
                      jnp.pad(W_neigh2, ((0, 0), (0, FH - C))),
                      row(jnp.pad(b2, (0, FH - C))))
  (P2,) = _segsum_sc(Y, src1, dst1, n, False)
  out64 = _tc_layer2b(S, P2, dgc)
  return out64[:, :C]


# speedup vs baseline: 1.1635x; 1.1635x over previous
import numpy as np
import jax
import jax.numpy as jnp
from jax import lax
from jax.experimental import pallas as pl
from jax.experimental.pallas import tpu as pltpu
from jax.experimental.pallas import tpu_sc as plsc

NC = 2
NS = 16
NWORK = NC * NS
WVE = 200
FH = 64

_SC_PARAMS = pltpu.CompilerParams(use_tc_tiling_on_sc=False)
_BN_K = float(1.0 / np.sqrt(1.0 + 1e-5))


def _segsum_sc(feat, src3, dst3, n_nodes, with_deg):
  dt = feat.dtype
  VL = 32 if dt == jnp.bfloat16 else 16
  F = feat.shape[1]
  NWIN = src3.shape[-2]
  ZCH = n_nodes // NS

  out_types = [jax.ShapeDtypeStruct((NC, n_nodes, F), dt)]
  if with_deg:
    out_types.append(jax.ShapeDtypeStruct((NC, n_nodes), jnp.float32))

  scratch = [
      pltpu.VMEM((NWIN, WVE), jnp.int32),
      pltpu.VMEM((NWIN, WVE), jnp.int32),
      pltpu.VMEM((WVE, F), dt),
      pltpu.VMEM((WVE, F), dt),
      pltpu.VMEM((WVE, F), dt),
      pltpu.VMEM((WVE, F), dt),
      pltpu.VMEM_SHARED((n_nodes, F), dt),
      pltpu.SemaphoreType.DMA,
      pltpu.SemaphoreType.DMA,
  ]
  DCH = 640
  if with_deg:
    scratch.append(pltpu.VMEM((DCH,), jnp.float32))
    scratch.append(pltpu.VMEM_SHARED((n_nodes,), jnp.float32))

  mesh = plsc.VectorSubcoreMesh(core_axis_name="c", subcore_axis_name="s")

  def body(feat_hbm, src_hbm, dst_hbm, *rest):
    if with_deg:
      (out_hbm, deg_hbm, src_v, dst_v, buf0, buf1, buf2, buf3, acc, gsem,
       ssem, ones_v, dacc) = rest
    else:
      out_hbm, src_v, dst_v, buf0, buf1, buf2, buf3, acc, gsem, ssem = rest
    bufs = (buf0, buf1, buf2, buf3)
    cid = lax.axis_index("c")
    tid = lax.axis_index("s")

    wid = tid * NC + cid
    pltpu.sync_copy(src_hbm.at[wid], src_v)
    pltpu.sync_copy(dst_hbm.at[wid], dst_v)

    @pl.loop(0, WVE)
    def _(r):
      for cc in range(F // VL):
        buf0[r, pl.ds(cc * VL, VL)] = jnp.zeros((VL,), dt)

    base = tid * ZCH
    nfull, rem = ZCH // WVE, ZCH % WVE
    for k in range(nfull):
      pltpu.sync_copy(buf0, acc.at[pl.ds(base + k * WVE, WVE)])
    if rem:
      pltpu.sync_copy(buf0.at[pl.ds(0, rem)],
                      acc.at[pl.ds(base + nfull * WVE, rem)])

    if with_deg:
      dlast = n_nodes - (NS - 1) * DCH

      @pl.loop(0, DCH // 16)
      def _(r):
        ones_v[pl.ds(r * 16, 16)] = jnp.zeros((16,), jnp.float32)

      @pl.when(tid < NS - 1)
      def _():
        pltpu.sync_copy(ones_v, dacc.at[pl.ds(tid * DCH, DCH)])

      @pl.when(tid == NS - 1)
      def _():
        pltpu.sync_copy(ones_v.at[pl.ds(0, dlast)],
                        dacc.at[pl.ds((NS - 1) * DCH, dlast)])

      @pl.loop(0, DCH // 16)
      def _(r):
        ones_v[pl.ds(r * 16, 16)] = jnp.ones((16,), jnp.float32)

    plsc.subcore_barrier()

    def gstart(w, b):
      pltpu.async_copy(feat_hbm.at[src_v.at[w]], bufs[b], gsem)

    def gwait(b):
      pltpu.make_async_copy(feat_hbm.at[src_v.at[0]], bufs[b], gsem).wait()

    def sstart(w, b):
      pltpu.async_copy(bufs[b], acc.at[dst_v.at[w]], ssem, add=True)
      if with_deg:
        pltpu.sync_copy(ones_v.at[pl.ds(0, WVE)], dacc.at[dst_v.at[w]],
                        add=True)

    def swait(b):
      pltpu.make_async_copy(bufs[b], acc.at[dst_v.at[0]], ssem).wait()

    assert NWIN >= 8
    K = (NWIN - 8) // 4
    post0 = 4 + 4 * K

    gstart(0, 0)
    gstart(1, 1)
    for w in range(4):
      b = w % 4
      gwait(b)
      if w >= 2:
        swait(w - 2)
      sstart(w, b)
      gstart(w + 2, (w + 2) % 4)

    @pl.loop(4, post0, step=4)
    def _(j):
      for u in range(4):
        gwait(u)
        swait((u + 2) % 4)
        sstart(j + u, u)
        gstart(j + u + 2, (u + 2) % 4)

    for w in range(post0, NWIN):
      b = w % 4
      gwait(b)
      swait((w - 2) % 4)
      sstart(w, b)
      if w + 2 < NWIN:
        gstart(w + 2, (w + 2) % 4)
    swait((NWIN - 2) % 4)
    swait((NWIN - 1) % 4)

    plsc.subcore_barrier()

    pltpu.sync_copy(acc.at[pl.ds(base, ZCH)],
                    out_hbm.at[cid, pl.ds(base, ZCH)])
    if with_deg:
      @pl.when(tid == 0)
      def _():
        pltpu.sync_copy(dacc, deg_hbm.at[cid])

  k = pl.kernel(body, out_type=tuple(out_types), mesh=mesh,
                scratch_types=scratch, compiler_params=_SC_PARAMS)
  res = k(feat, src3, dst3)
  return res if with_deg else (res[0],)


def _tc_l0_pre(x, plm, ws, b, g, bb, BR=1000):
  n, F = x.shape

  def body(xr, pr, wsr, br, gr, bbr, outr):
    f32 = jnp.float32
    t = (jnp.dot(xr[...].astype(f32), wsr[...], preferred_element_type=f32)
         + br[...])
    outr[...] = ((t + pr[...]) * gr[...] * _BN_K + 2.0 * bbr[...])

  return pl.pallas_call(
      body, grid=(n // BR,),
      in_specs=[
          pl.BlockSpec((BR, F), lambda i: (i, 0)),
          pl.BlockSpec((BR, F), lambda i: (i, 0)),
          pl.BlockSpec((F, F), lambda i: (0, 0)),
          pl.BlockSpec((1, F), lambda i: (0, 0)),
          pl.BlockSpec((1, F), lambda i: (0, 0)),
          pl.BlockSpec((1, F), lambda i: (0, 0)),
      ],
      out_specs=pl.BlockSpec((BR, F), lambda i: (i, 0)),
      out_shape=jax.ShapeDtypeStruct((n, F), jnp.float32),
  )(x, plm, ws, b, g, bb)


def _tc_l0_post(u, p, dg, wn, g, BR=1000):
  n, F = u.shape

  def body(ur, ppr, dgr, wnr, gr, outr):
    f32 = jnp.float32
    dm = jnp.maximum(dgr[0] + dgr[1], 1.0)
    pm = (ppr[0].astype(f32) + ppr[1].astype(f32)) / dm
    v = jnp.dot(pm, wnr[...], preferred_element_type=f32) * gr[...] * _BN_K
    outr[...] = jnp.maximum(ur[...] + v, 0.0).astype(outr.dtype)

  return pl.pallas_call(
      body, grid=(n // BR,),
      in_specs=[
          pl.BlockSpec((BR, F), lambda i: (i, 0)),
          pl.BlockSpec((NC, BR, F), lambda i: (0, i, 0)),
          pl.BlockSpec((NC, BR, 1), lambda i: (0, i, 0)),
          pl.BlockSpec((F, F), lambda i: (0, 0)),
          pl.BlockSpec((1, F), lambda i: (0, 0)),
      ],
      out_specs=pl.BlockSpec((BR, F), lambda i: (i, 0)),
      out_shape=jax.ShapeDtypeStruct((n, F), jnp.bfloat16),
  )(u, p, dg, wn, g)


def _tc_layer01(x, plm, p, dg, ws, wn, b, g, bb, BR=1000):
  n, F = x.shape
  Fo = ws.shape[1]
  have_plm = plm is not None

  def body(*refs):
    if have_plm:
      xr, pr, ppr, dgr, wsr, wnr, br, gr, bbr, outr = refs
    else:
      xr, ppr, dgr, wsr, wnr, br, gr, bbr, outr = refs
    f32 = jnp.float32
    dm = jnp.maximum(dgr[0] + dgr[1], 1.0)
    pm = (ppr[0].astype(f32) + ppr[1].astype(f32)) / dm
    t = (jnp.dot(xr[...].astype(f32), wsr[...], preferred_element_type=f32)
         + jnp.dot(pm, wnr[...], preferred_element_type=f32)
         + br[...])
    t = t * gr[...] * _BN_K + bbr[...]
    if have_plm:
      t = t + pr[...] * gr[...] * _BN_K + bbr[...]
    outr[...] = jnp.maximum(t, 0.0).astype(outr.dtype)

  in_specs = [pl.BlockSpec((BR, F), lambda i: (i, 0))]
  if have_plm:
    in_specs.append(pl.BlockSpec((BR, F), lambda i: (i, 0)))
  in_specs += [
      pl.BlockSpec((NC, BR, F), lambda i: (0, i, 0)),
      pl.BlockSpec((NC, BR, 1), lambda i: (0, i, 0)),
      pl.BlockSpec((F, Fo), lambda i: (0, 0)),
      pl.BlockSpec((F, Fo), lambda i: (0, 0)),
      pl.BlockSpec((1, Fo), lambda i: (0, 0)),
      pl.BlockSpec((1, Fo), lambda i: (0, 0)),
      pl.BlockSpec((1, Fo), lambda i: (0, 0)),
  ]
  args = [x] + ([plm] if have_plm else []) + [p, dg, ws, wn, b, g, bb]
  return pl.pallas_call(
      body, grid=(n // BR,), in_specs=in_specs,
      out_specs=pl.BlockSpec((BR, Fo), lambda i: (i, 0)),
      out_shape=jax.ShapeDtypeStruct((n, Fo), x.dtype),
  )(*args)


def _tc_layer12a(x, p, dg, ws, wn, b, g, bb, wsp, wnp, bp, BR=1000):
  n, F = x.shape
  Fo = wsp.shape[1]

  def body(xr, ppr, dgr, wsr, wnr, br, gr, bbr, wspr, wnpr, bpr, sr, yr):
    f32 = jnp.float32
    dm = jnp.maximum(dgr[0] + dgr[1], 1.0)
    pm = (ppr[0].astype(f32) + ppr[1].astype(f32)) / dm
    t = (jnp.dot(xr[...].astype(f32), wsr[...], preferred_element_type=f32)
         + jnp.dot(pm, wnr[...], preferred_element_type=f32)
         + br[...])
    h2 = jnp.maximum(t * gr[...] * _BN_K + bbr[...], 0.0)
    sr[...] = (jnp.dot(h2, wspr[...], preferred_element_type=f32) + bpr[...])
    yr[...] = jnp.dot(h2, wnpr[...],
                      preferred_element_type=f32).astype(yr.dtype)

  full = lambda shape: pl.BlockSpec(shape, lambda i: tuple(0 for _ in shape))
  return pl.pallas_call(
      body, grid=(n // BR,),
      in_specs=[
          pl.BlockSpec((BR, F), lambda i: (i, 0)),
          pl.BlockSpec((NC, BR, F), lambda i: (0, i, 0)),
          pl.BlockSpec((NC, BR, 1), lambda i: (0, i, 0)),
          full((F, F)), full((F, F)), full((1, F)), full((1, F)),
          full((1, F)), full((F, Fo)), full((F, Fo)), full((1, Fo)),
      ],
      out_specs=[pl.BlockSpec((BR, Fo), lambda i: (i, 0)),
                 pl.BlockSpec((BR, Fo), lambda i: (i, 0))],
      out_shape=[jax.ShapeDtypeStruct((n, Fo), jnp.float32),
                 jax.ShapeDtypeStruct((n, Fo), x.dtype)],
  )(x, p, dg, ws, wn, b, g, bb, wsp, wnp, bp)


def _tc_layer2b(s, p, dg, BR=1000):
  n, Fo = s.shape

  def body(sr, ppr, dgr, outr):
    dm = jnp.maximum(dgr[0] + dgr[1], 1.0)
    outr[...] = sr[...] + (ppr[0].astype(jnp.float32)
                           + ppr[1].astype(jnp.float32)) / dm

  return pl.pallas_call(
      body, grid=(n // BR,),
      in_specs=[
          pl.BlockSpec((BR, Fo), lambda i: (i, 0)),
          pl.BlockSpec((NC, BR, Fo), lambda i: (0, i, 0)),
          pl.BlockSpec((NC, BR, 1), lambda i: (0, i, 0)),
      ],
      out_specs=pl.BlockSpec((BR, Fo), lambda i: (i, 0)),
      out_shape=jax.ShapeDtypeStruct((n, Fo), jnp.float32),
  )(s, p, dg)


def kernel(edge_index, LLM_feat, PLM_feat, W_self0, W_neigh0, b0,
           W_self1, W_neigh1, b1, W_self2, W_neigh2, b2,
           bn0_g, bn0_b, bn1_g, bn1_b):
  n, D = LLM_feat.shape
  E = edge_index.shape[1]
  C = W_self2.shape[1]
  EPW = E // NWORK
  NWIN = EPW // WVE
  assert E == NWORK * NWIN * WVE and n % NS == 0

  src1 = edge_index[0].astype(jnp.int32).reshape(NWORK, NWIN, WVE)
  dst1 = edge_index[1].astype(jnp.int32).reshape(NWORK, NWIN, WVE)

  row = lambda v: v.reshape(1, -1)

  llm_b = LLM_feat.astype(jnp.bfloat16)
  P0, Dg = _segsum_sc(llm_b, src1, dst1, n, True)
  dgc = Dg.reshape(NC, n, 1)
  U = _tc_l0_pre(llm_b, PLM_feat, W_self0, row(b0), row(bn0_g), row(bn0_b))
  h1 = _tc_l0_post(U, P0, dgc, W_neigh0, row(bn0_g))
  (P1,) = _segsum_sc(h1, src1, dst1, n, False)
  S, Y = _tc_layer12a(h1, P1, dgc, W_self1, W_neigh1,
                      row(b1), row(bn1_g), row(bn1_b),
                      jnp.pad(W_self2, ((0, 0), (0, FH - C))),
                      jnp.pad(W_neigh2, ((0, 0), (0, FH - C))),
                      row(jnp.pad(b2, (0, FH - C))))
  (P2,) = _segsum_sc(Y, src1, dst1, n, False)
  out64 = _tc_layer2b(S, P2, dgc)
  return out64[:, :C]
